# baseline (speedup 1.0000x reference)
import functools

import jax
import jax.numpy as jnp
from jax import lax
from jax.experimental import pallas as pl
from jax.experimental.pallas import tpu as pltpu

_EMBED_DIM = 64
_D_MODEL = 2 * _EMBED_DIM
_HIDDEN = 2 * _D_MODEL
_STEPS = 4
_BQ = 1024
_BS = 64
_NUM_IDX = _BQ * 2 + _BS * 2


def _dot_t(a, b):
    return lax.dot_general(a, b, (((1,), (1,)), ((), ())),
                           preferred_element_type=jnp.float32)


def _body(idx_ref, table_ref, w1_ref, b1_ref, w2_ref, b2_ref, lng_ref,
          lnb_ref, wih_ref, whh_ref, bih_ref, bhh_ref, out_ref,
          rows_v, sem):
    def _fire(i, _):
        pltpu.make_async_copy(table_ref.at[idx_ref[i]], rows_v.at[i],
                              sem).start()
        return _

    lax.fori_loop(0, _NUM_IDX, _fire, 0, unroll=8)

    pltpu.make_async_copy(table_ref.at[pl.ds(0, _NUM_IDX)], rows_v,
                          sem).wait()

    q = jnp.concatenate([rows_v[:_BQ], rows_v[_BQ:2 * _BQ]], axis=1)
    s = jnp.concatenate([rows_v[2 * _BQ:2 * _BQ + _BS],
                         rows_v[2 * _BQ + _BS:_NUM_IDX]], axis=1)

    hid = jnp.maximum(_dot_t(s, w1_ref[...]) + b1_ref[...], 0.0)
    y = _dot_t(hid, w2_ref[...]) + b2_ref[...] + s
    mu = jnp.mean(y, axis=-1, keepdims=True)
    var = jnp.mean((y - mu) * (y - mu), axis=-1, keepdims=True)
    sg = lng_ref[...] * (y - mu) * lax.rsqrt(var + 1e-5) + lnb_ref[...]

    xg = _dot_t(q, wih_ref[...]) + bih_ref[...] + bhh_ref[...]
    h_r = jnp.zeros((_BQ, _HIDDEN), jnp.float32)
    c = jnp.zeros((_BQ, _HIDDEN), jnp.float32)
    h = q
    for _ in range(_STEPS):
        gates = xg + _dot_t(h_r, whh_ref[...])
        i_g = jax.nn.sigmoid(gates[:, :_HIDDEN])
        f_g = jax.nn.sigmoid(gates[:, _HIDDEN:2 * _HIDDEN])
        g_g = jnp.tanh(gates[:, 2 * _HIDDEN:3 * _HIDDEN])
        o_g = jax.nn.sigmoid(gates[:, 3 * _HIDDEN:])
        c = f_g * c + i_g * g_g
        h_full = o_g * jnp.tanh(c)
        h = q + h_full[:, :_D_MODEL]
        logits = _dot_t(h, sg)
        m = jnp.max(logits, axis=1, keepdims=True)
        e = jnp.exp(logits - m)
        attn = e / jnp.sum(e, axis=1, keepdims=True)
        r = jnp.dot(attn, sg, preferred_element_type=jnp.float32)
        h_r = jnp.concatenate([h, r], axis=1)

    out_ref[...] = _dot_t(h, sg)


def _call(interpret=False):
    grid_spec = pltpu.PrefetchScalarGridSpec(
        num_scalar_prefetch=1,
        grid=(1,),
        in_specs=[
            pl.BlockSpec(memory_space=pl.ANY),
        ] + [pl.BlockSpec(memory_space=pltpu.MemorySpace.VMEM)] * 10,
        out_specs=pl.BlockSpec(memory_space=pltpu.MemorySpace.VMEM),
        scratch_shapes=[
            pltpu.VMEM((_NUM_IDX, _EMBED_DIM), jnp.float32),
            pltpu.SemaphoreType.DMA,
        ],
    )
    return pl.pallas_call(
        _body,
        grid_spec=grid_spec,
        out_shape=jax.ShapeDtypeStruct((_BQ, _BS), jnp.float32),
        interpret=interpret,
    )


def kernel(query, support, symbol_emb, W1, b1, W2, b2, ln_g, ln_b,
           W_ih, W_hh, b_ih, b_hh):
    idx = jnp.concatenate([
        query[:, 0].astype(jnp.int32),
        query[:, 1].astype(jnp.int32),
        support[:, 0].astype(jnp.int32),
        support[:, 1].astype(jnp.int32),
    ])
    return _call()(
        idx, symbol_emb,
        W1, b1.reshape(1, -1), W2, b2.reshape(1, -1),
        ln_g.reshape(1, -1), ln_b.reshape(1, -1), W_ih, W_hh,
        b_ih.reshape(1, -1), b_hh.reshape(1, -1))


# speedup vs baseline: 1.0037x; 1.0037x over previous
import functools

import jax
import jax.numpy as jnp
from jax import lax
from jax.experimental import pallas as pl
from jax.experimental.pallas import tpu as pltpu

_EMBED_DIM = 64
_D_MODEL = 2 * _EMBED_DIM
_HIDDEN = 2 * _D_MODEL
_STEPS = 4
_BQ = 1024
_BS = 64
_NUM_IDX = _BQ * 2 + _BS * 2


def _dot_t(a, b):
    return lax.dot_general(a, b, (((1,), (1,)), ((), ())),
                           preferred_element_type=jnp.float32)


def _dot_t16(a, b):
    return lax.dot_general(a.astype(jnp.bfloat16), b.astype(jnp.bfloat16),
                           (((1,), (1,)), ((), ())),
                           preferred_element_type=jnp.float32)


def _body(idx_ref, table_ref, w1_ref, b1_ref, w2_ref, b2_ref, lng_ref,
          lnb_ref, wih_ref, whh_ref, bih_ref, bhh_ref, out_ref,
          rows_v, sem):
    def _fire(i, _):
        pltpu.make_async_copy(table_ref.at[idx_ref[i]], rows_v.at[i],
                              sem).start()
        return _

    lax.fori_loop(0, _NUM_IDX, _fire, 0, unroll=8)

    pltpu.make_async_copy(table_ref.at[pl.ds(0, _NUM_IDX)], rows_v,
                          sem).wait()

    q = jnp.concatenate([rows_v[:_BQ], rows_v[_BQ:2 * _BQ]], axis=1)
    s = jnp.concatenate([rows_v[2 * _BQ:2 * _BQ + _BS],
                         rows_v[2 * _BQ + _BS:_NUM_IDX]], axis=1)

    hid = jnp.maximum(_dot_t(s, w1_ref[...]) + b1_ref[...], 0.0)
    y = _dot_t(hid, w2_ref[...]) + b2_ref[...] + s
    mu = jnp.mean(y, axis=-1, keepdims=True)
    var = jnp.mean((y - mu) * (y - mu), axis=-1, keepdims=True)
    sg = lng_ref[...] * (y - mu) * lax.rsqrt(var + 1e-5) + lnb_ref[...]

    xg = _dot_t16(q, wih_ref[...]) + bih_ref[...] + bhh_ref[...]
    h_r = jnp.zeros((_BQ, _HIDDEN), jnp.float32)
    c = jnp.zeros((_BQ, _HIDDEN), jnp.float32)
    h = q
    for _ in range(_STEPS):
        gates = xg + _dot_t16(h_r, whh_ref[...])
        i_g = jax.nn.sigmoid(gates[:, :_HIDDEN])
        f_g = jax.nn.sigmoid(gates[:, _HIDDEN:2 * _HIDDEN])
        g_g = jnp.tanh(gates[:, 2 * _HIDDEN:3 * _HIDDEN])
        o_g = jax.nn.sigmoid(gates[:, 3 * _HIDDEN:])
        c = f_g * c + i_g * g_g
        h_full = o_g * jnp.tanh(c)
        h = q + h_full[:, :_D_MODEL]
        logits = _dot_t(h, sg)
        m = jnp.max(logits, axis=1, keepdims=True)
        e = jnp.exp(logits - m)
        attn = e / jnp.sum(e, axis=1, keepdims=True)
        r = jnp.dot(attn, sg, preferred_element_type=jnp.float32)
        h_r = jnp.concatenate([h, r], axis=1)

    out_ref[...] = _dot_t(h, sg)


def _call(interpret=False):
    grid_spec = pltpu.PrefetchScalarGridSpec(
        num_scalar_prefetch=1,
        grid=(1,),
        in_specs=[
            pl.BlockSpec(memory_space=pl.ANY),
        ] + [pl.BlockSpec(memory_space=pltpu.MemorySpace.VMEM)] * 10,
        out_specs=pl.BlockSpec(memory_space=pltpu.MemorySpace.VMEM),
        scratch_shapes=[
            pltpu.VMEM((_NUM_IDX, _EMBED_DIM), jnp.float32),
            pltpu.SemaphoreType.DMA,
        ],
    )
    return pl.pallas_call(
        _body,
        grid_spec=grid_spec,
        out_shape=jax.ShapeDtypeStruct((_BQ, _BS), jnp.float32),
        interpret=interpret,
    )


def kernel(query, support, symbol_emb, W1, b1, W2, b2, ln_g, ln_b,
           W_ih, W_hh, b_ih, b_hh):
    idx = jnp.concatenate([
        query[:, 0].astype(jnp.int32),
        query[:, 1].astype(jnp.int32),
        support[:, 0].astype(jnp.int32),
        support[:, 1].astype(jnp.int32),
    ])
    return _call()(
        idx, symbol_emb,
        W1, b1.reshape(1, -1), W2, b2.reshape(1, -1),
        ln_g.reshape(1, -1), ln_b.reshape(1, -1), W_ih, W_hh,
        b_ih.reshape(1, -1), b_hh.reshape(1, -1))
